# baseline (speedup 1.0000x reference)
import functools

import jax
import jax.numpy as jnp
from jax import lax
from jax.experimental import pallas as pl
from jax.experimental.pallas import tpu as pltpu
from jax.experimental.pallas import tpu_sc as plsc


def _gather_body(tableT_hbm, ids_hbm, outT_hbm, row_v, ids_v, o0_v, o1_v,
                 sem_r, sem_i, sem_o0, sem_o1, *,
                 num_cores, rows_per_w, oc, n_oc):
    wid = lax.axis_index("c") * 16 + lax.axis_index("s")
    obufs = (o0_v, o1_v)
    osems = (sem_o0, sem_o1)
    pending = [None, None]
    def start_row_copy(r):
        return [pltpu.async_copy(tableT_hbm.at[r], row_v, sem_r)]

    row_cps = start_row_copy(wid * rows_per_w)
    pltpu.async_copy(ids_hbm, ids_v, sem_i).wait()
    for cp in row_cps:
        cp.wait()

    for p in range(rows_per_w):
        row = wid * rows_per_w + p
        for c in range(n_oc):
            ob = obufs[c % 2]
            if pending[c % 2] is not None:
                pending[c % 2].wait()
                pending[c % 2] = None
            base = c * oc

            @plsc.parallel_loop(0, oc // 16, 1, unroll=8)
            def gather_iter(j):
                idx = ids_v[pl.ds(base + j * 16, 16)]
                ob[pl.ds(j * 16, 16)] = plsc.load_gather(row_v, [idx])

            if c == n_oc - 1 and p + 1 < rows_per_w:
                row_cps = start_row_copy(row + 1)
            pending[c % 2] = pltpu.async_copy(
                ob, outT_hbm.at[row, pl.ds(base, oc)], osems[c % 2])
        if p + 1 < rows_per_w:
            for cp in row_cps:
                cp.wait()
    for q in range(2):
        if pending[q] is not None:
            pending[q].wait()


def kernel(kernel, ids):
    table = kernel
    V, D = table.shape
    B = ids.shape[0]
    ids32 = ids.astype(jnp.int32)
    tableT = table.T

    info = plsc.get_sparse_core_info()
    nw = info.num_cores * info.num_subcores
    rows_per_w = D // nw
    oc = 4096
    n_oc = B // oc

    mesh = plsc.VectorSubcoreMesh(core_axis_name="c", subcore_axis_name="s")
    body = functools.partial(_gather_body, num_cores=info.num_cores,
                             rows_per_w=rows_per_w, oc=oc, n_oc=n_oc)
    run = pl.kernel(
        body,
        mesh=mesh,
        out_type=jax.ShapeDtypeStruct((D, B), jnp.float32),
        scratch_types=[
            pltpu.VMEM((V,), jnp.float32),
            pltpu.VMEM((B,), jnp.int32),
            pltpu.VMEM((oc,), jnp.float32),
            pltpu.VMEM((oc,), jnp.float32),
            pltpu.SemaphoreType.DMA,
            pltpu.SemaphoreType.DMA,
            pltpu.SemaphoreType.DMA,
            pltpu.SemaphoreType.DMA,
        ],
        compiler_params=pltpu.CompilerParams(needs_layout_passes=False),
    )
    outT = run(tableT, ids32)
    return outT.T


# speedup vs baseline: 1.0037x; 1.0037x over previous
import functools

import jax
import jax.numpy as jnp
from jax import lax
from jax.experimental import pallas as pl
from jax.experimental.pallas import tpu as pltpu
from jax.experimental.pallas import tpu_sc as plsc


def _gather_body(tableT_hbm, ids_hbm, outT_hbm, row_v, ids_v, o0_v, o1_v,
                 sem_r, sem_i, sem_o0, sem_o1, *,
                 num_subcores, rows_per_w, oc, n_oc):
    wid = lax.axis_index("c") * num_subcores + lax.axis_index("s")
    obufs = (o0_v, o1_v)
    osems = (sem_o0, sem_o1)
    pending = [None, None]

    row_cp = pltpu.async_copy(tableT_hbm.at[wid * rows_per_w], row_v, sem_r)
    pltpu.async_copy(ids_hbm, ids_v, sem_i).wait()
    row_cp.wait()

    for p in range(rows_per_w):
        row = wid * rows_per_w + p
        for c in range(n_oc):
            ob = obufs[c % 2]
            if pending[c % 2] is not None:
                pending[c % 2].wait()
                pending[c % 2] = None
            base = c * oc

            @plsc.parallel_loop(0, oc // 16, 1, unroll=8)
            def gather_iter(j):
                idx = ids_v[pl.ds(base + j * 16, 16)]
                ob[pl.ds(j * 16, 16)] = plsc.load_gather(row_v, [idx])

            if c == n_oc - 1 and p + 1 < rows_per_w:
                row_cp = pltpu.async_copy(
                    tableT_hbm.at[row + 1], row_v, sem_r)
            pending[c % 2] = pltpu.async_copy(
                ob, outT_hbm.at[row, pl.ds(base, oc)], osems[c % 2])
        if p + 1 < rows_per_w:
            row_cp.wait()
    for q in range(2):
        if pending[q] is not None:
            pending[q].wait()


def kernel(kernel, ids):
    table = kernel
    V, D = table.shape
    B = ids.shape[0]
    ids32 = ids.astype(jnp.int32)
    tableT = table.T

    info = plsc.get_sparse_core_info()
    nw = info.num_cores * info.num_subcores
    rows_per_w = D // nw
    oc = 4096
    n_oc = B // oc

    mesh = plsc.VectorSubcoreMesh(core_axis_name="c", subcore_axis_name="s")
    body = functools.partial(_gather_body, num_subcores=info.num_subcores,
                             rows_per_w=rows_per_w, oc=oc, n_oc=n_oc)
    run = pl.kernel(
        body,
        mesh=mesh,
        out_type=jax.ShapeDtypeStruct((D, B), jnp.float32),
        scratch_types=[
            pltpu.VMEM((V,), jnp.float32),
            pltpu.VMEM((B,), jnp.int32),
            pltpu.VMEM((oc,), jnp.float32),
            pltpu.VMEM((oc,), jnp.float32),
            pltpu.SemaphoreType.DMA,
            pltpu.SemaphoreType.DMA,
            pltpu.SemaphoreType.DMA,
            pltpu.SemaphoreType.DMA,
        ],
        compiler_params=pltpu.CompilerParams(needs_layout_passes=False),
    )
    outT = run(tableT, ids32)
    return outT.T
